# baseline (speedup 1.0000x reference)
import functools
import math

import jax
import jax.numpy as jnp
from jax import lax
from jax.experimental import pallas as pl
from jax.experimental.pallas import tpu as pltpu
from jax.experimental.pallas import tpu_sc as plsc

N, E, DIN, DE, H, K, Z = 10000, 320000, 128, 16, 128, 10, 32

BN = 1000
BE = 1600
_NPARTS = 5
_PE = E // _NPARTS

_NCORES = 2
_NSUB = 16
_NW = _NCORES * _NSUB
_CHUNK = 80
_GROUP = 5
_GC = _CHUNK * _GROUP
_NP = 10240
_GCH = 40
_GSET = 200

def _sc_mesh():
    return plsc.VectorSubcoreMesh(core_axis_name="c", subcore_axis_name="s")


def _ln(h, g, b):
    mu = jnp.mean(h, axis=-1, keepdims=True)
    var = jnp.mean((h - mu) ** 2, axis=-1, keepdims=True)
    return (h - mu) * lax.rsqrt(var + 1e-5) * g + b


def _mlp_ln_body(x_ref, w1_ref, b1_ref, w2_ref, b2_ref, g_ref, bb_ref, o_ref):
    h = jnp.maximum(jnp.dot(x_ref[...], w1_ref[...],
                            preferred_element_type=jnp.float32) + b1_ref[...], 0.0)
    h = jnp.dot(h, w2_ref[...], preferred_element_type=jnp.float32) + b2_ref[...]
    o_ref[...] = _ln(h, g_ref[...], bb_ref[...])


def _mlp_ln(xa, p, bm, row0=0, rows=None):
    r, din = xa.shape
    rows = r if rows is None else rows
    blk0 = row0 // bm
    dh = p["W1"].shape[1]
    dout = p["W2"].shape[1]
    full = lambda shape: pl.BlockSpec(shape, lambda i: (0, 0))
    return pl.pallas_call(
        _mlp_ln_body,
        grid=(rows // bm,),
        in_specs=[
            pl.BlockSpec((bm, din), lambda i: (i + blk0, 0)),
            full((din, dh)), full((1, dh)), full((dh, dout)), full((1, dout)),
            full((1, dout)), full((1, dout)),
        ],
        out_specs=pl.BlockSpec((bm, dout), lambda i: (i, 0)),
        out_shape=jax.ShapeDtypeStruct((rows, dout), jnp.float32),
    )(xa, p["W1"], p["b1"].reshape(1, dh), p["W2"], p["b2"].reshape(1, dout),
      p["g"].reshape(1, dout), p["b"].reshape(1, dout))


def _pack_bf16(x):
    u = lax.bitcast_convert_type(x, jnp.int32) + jnp.int32(0x8000)
    lo = lax.shift_right_logical(u[:, :H // 2], 16)
    hi = jnp.bitwise_and(u[:, H // 2:], jnp.int32(-65536))
    return jnp.bitwise_or(lo, hi)


def _unpack_bf16(x):
    lo = lax.bitcast_convert_type(lax.shift_left(x, 16), jnp.float32)
    hi = lax.bitcast_convert_type(
        jnp.bitwise_and(x, jnp.int32(-65536)), jnp.float32)
    return jnp.concatenate([lo, hi], axis=-1)


def _proj_body(h_ref, ws_ref, wd_ref, ps_ref, pd_ref):
    h = h_ref[...]
    ps_ref[...] = _pack_bf16(jnp.dot(h, ws_ref[...],
                                     preferred_element_type=jnp.float32))
    pd_ref[...] = _pack_bf16(jnp.dot(h, wd_ref[...],
                                     preferred_element_type=jnp.float32))


def _proj(h, ws, wd):
    full = lambda shape: pl.BlockSpec(shape, lambda i: (0, 0))
    out = jax.ShapeDtypeStruct((N, H // 2), jnp.int32)
    return pl.pallas_call(
        _proj_body,
        grid=(N // BN,),
        in_specs=[pl.BlockSpec((BN, H), lambda i: (i, 0)), full((H, H)), full((H, H))],
        out_specs=[pl.BlockSpec((BN, H // 2), lambda i: (i, 0))] * 2,
        out_shape=[out, out],
    )(h, ws, wd)


def _edge_body(gs_ref, gd_ref, e_ref, we_ref, b1_ref, w2_ref, b2_ref, g_ref,
               bb_ref, en_ref, eo_ref):
    e = e_ref[...]
    t = (_unpack_bf16(gs_ref[...]) + _unpack_bf16(gd_ref[...])
         + jnp.dot(e, we_ref[...], preferred_element_type=jnp.float32)
         + b1_ref[...])
    u = jnp.dot(jnp.maximum(t, 0.0), w2_ref[...],
                preferred_element_type=jnp.float32) + b2_ref[...]
    en = _ln(u, g_ref[...], bb_ref[...])
    en_ref[...] = en
    if eo_ref is not None:
        eo_ref[...] = e + en


def _edge_update(gs, gd, e, we, p, residual):
    ne = gs.shape[0]
    full = lambda shape: pl.BlockSpec(shape, lambda i: (0, 0))
    row = pl.BlockSpec((BE, H), lambda i: (i, 0))
    rowp = pl.BlockSpec((BE, H // 2), lambda i: (i, 0))
    out = jax.ShapeDtypeStruct((ne, H), jnp.float32)
    body = _edge_body if residual else functools.partial(_edge_body, eo_ref=None)
    return pl.pallas_call(
        body,
        grid=(ne // BE,),
        in_specs=[rowp, rowp, row, full((H, H)), full((1, H)), full((H, H)),
                  full((1, H)), full((1, H)), full((1, H))],
        out_specs=[row, row] if residual else [row],
        out_shape=[out, out] if residual else [out],
    )(gs, gd, e, we, p["b1"].reshape(1, H), p["W2"], p["b2"].reshape(1, H),
      p["g"].reshape(1, H), p["b"].reshape(1, H))


def _node_body(h_ref, a0_ref, a1_ref, wh_ref, wa_ref, b1_ref, w2_ref, b2_ref,
               g_ref, bb_ref, o_ref):
    h = h_ref[...]
    agg = a0_ref[...] + a1_ref[...]
    t = (jnp.dot(h, wh_ref[...], preferred_element_type=jnp.float32)
         + jnp.dot(agg, wa_ref[...], preferred_element_type=jnp.float32)
         + b1_ref[...])
    u = jnp.dot(jnp.maximum(t, 0.0), w2_ref[...],
                preferred_element_type=jnp.float32) + b2_ref[...]
    o_ref[...] = h + _ln(u, g_ref[...], bb_ref[...])


def _node_update(h, a0, a1, p):
    full = lambda shape: pl.BlockSpec(shape, lambda i: (0, 0))
    row = pl.BlockSpec((BN, H), lambda i: (i, 0))
    wh, wa = p["W1"][:H], p["W1"][H:]
    return pl.pallas_call(
        _node_body,
        grid=(N // BN,),
        in_specs=[row, row, row, full((H, H)), full((H, H)), full((1, H)),
                  full((H, H)), full((1, H)), full((1, H)), full((1, H))],
        out_specs=row,
        out_shape=jax.ShapeDtypeStruct((N, H), jnp.float32),
    )(h, a0, a1, wh, wa, p["b1"].reshape(1, H), p["W2"], p["b2"].reshape(1, H),
      p["g"].reshape(1, H), p["b"].reshape(1, H))


_DOUT = K * (1 + 2 * Z)


def _pool_body(h_ref, gw_ref, w1_ref, b1_ref, w2_ref, b2_ref, mask_ref, o_ref,
               m_ref, s_ref, p_ref):
    i = pl.program_id(0)

    @pl.when(i == 0)
    def _():
        m_ref[0] = -1e30
        s_ref[0] = 0.0
        p_ref[...] = jnp.zeros_like(p_ref)

    h = h_ref[...]
    sloc = jnp.sum(h * gw_ref[...], axis=-1, keepdims=True)
    m_old = m_ref[0]
    m_new = jnp.maximum(m_old, jnp.max(sloc))
    c = jnp.exp(m_old - m_new)
    w = jnp.exp(sloc - m_new)
    s_ref[0] = s_ref[0] * c + jnp.sum(w)
    p_ref[...] = p_ref[...] * c + jnp.sum(w * h, axis=0, keepdims=True)
    m_ref[0] = m_new

    @pl.when(i == pl.num_programs(0) - 1)
    def _():
        pooled = p_ref[...] / s_ref[0]
        hh = jnp.maximum(jnp.dot(pooled, w1_ref[...],
                                 preferred_element_type=jnp.float32) + b1_ref[...], 0.0)
        raw = jnp.dot(hh, w2_ref[...],
                      preferred_element_type=jnp.float32) + b2_ref[...]
        o_ref[...] = jnp.where(mask_ref[...] > 0.0,
                               jnp.clip(raw, math.log(0.05), 5.0), raw)


def _pool_head(h, gw_row, p, mask):
    full = lambda shape: pl.BlockSpec(shape, lambda i: (0, 0))
    return pl.pallas_call(
        _pool_body,
        grid=(N // BN,),
        in_specs=[pl.BlockSpec((BN, H), lambda i: (i, 0)), full((1, H)),
                  full((H, H)), full((1, H)), full((H, _DOUT)), full((1, _DOUT)),
                  full((1, _DOUT))],
        out_specs=full((1, _DOUT)),
        out_shape=jax.ShapeDtypeStruct((1, _DOUT), jnp.float32),
        scratch_shapes=[pltpu.SMEM((1,), jnp.float32),
                        pltpu.SMEM((1,), jnp.float32),
                        pltpu.VMEM((1, H), jnp.float32)],
    )(h, gw_row, p["W1"], p["b1"].reshape(1, H), p["W2"],
      p["b2"].reshape(1, _DOUT), mask)


def _sc_gather_pair(ps, pd, src, dst, start, ne):
    out = jax.ShapeDtypeStruct((ne, H // 2), jnp.int32)
    nchunks = ne // _GC
    per_w = nchunks // _NW

    @functools.partial(
        pl.kernel,
        out_type=[out, out],
        mesh=_sc_mesh(),
        compiler_params=pltpu.CompilerParams(use_tc_tiling_on_sc=False),
        scratch_types=[
            pltpu.VMEM((_GC,), jnp.int32),
            pltpu.VMEM((_GC,), jnp.int32),
            pltpu.VMEM((_GC,), jnp.int32),
            pltpu.VMEM((_GC,), jnp.int32),
            pltpu.VMEM((_GC, H // 2), jnp.int32),
            pltpu.VMEM((_GC, H // 2), jnp.int32),
            pltpu.VMEM((_GC, H // 2), jnp.int32),
            pltpu.VMEM((_GC, H // 2), jnp.int32),
            pltpu.SemaphoreType.DMA,
            pltpu.SemaphoreType.DMA,
            pltpu.SemaphoreType.DMA,
            pltpu.SemaphoreType.DMA,
        ],
    )
    def k(ps_hbm, pd_hbm, src_hbm, dst_hbm, gs_hbm, gd_hbm,
          is0, id0, is1, id1, a0, b0, a1, b1, g0, g1, w0, w1):
        wid = lax.axis_index("s") * _NCORES + lax.axis_index("c")
        sets = ((is0, id0, a0, b0, g0, w0), (is1, id1, a1, b1, g1, w1))

        def fire(off, s):
            isv, idv, a_v, b_v, gsem, _ = sets[s]
            pltpu.sync_copy(src_hbm.at[pl.ds(start + off, _GC)], isv)
            pltpu.sync_copy(dst_hbm.at[pl.ds(start + off, _GC)], idv)
            hs = []
            for j in range(_GROUP):
                sl = pl.ds(j * _CHUNK, _CHUNK)
                hs.append(pltpu.async_copy(ps_hbm.at[isv.at[sl]], a_v.at[sl], gsem))
                hs.append(pltpu.async_copy(pd_hbm.at[idv.at[sl]], b_v.at[sl], gsem))
            return hs

        def writeback(off, s, wait_handles):
            _, _, a_v, b_v, _, wsem = sets[s]
            for h in wait_handles:
                h.wait()
            pltpu.async_copy(a_v, gs_hbm.at[pl.ds(off, _GC)], wsem)
            pltpu.async_copy(b_v, gd_hbm.at[pl.ds(off, _GC)], wsem)

        def drain_wb(s):
            _, _, a_v, b_v, _, wsem = sets[s]
            pltpu.make_async_copy(a_v, gs_hbm.at[pl.ds(0, _GC)], wsem).wait()
            pltpu.make_async_copy(b_v, gd_hbm.at[pl.ds(0, _GC)], wsem).wait()

        offs, hss = [], []
        for t in range(per_w):
            s = t % 2
            if t >= 2:
                drain_wb(s)
            off = (wid + t * _NW) * _GC
            hs = fire(off, s)
            if t >= 1:
                writeback(offs[t - 1], (t - 1) % 2, hss[t - 1])
            offs.append(off)
            hss.append(hs)
        writeback(offs[-1], (per_w - 1) % 2, hss[-1])
        drain_wb(0)
        drain_wb(1)

    return k(ps, pd, src, dst)


def _sc_segsum(en_parts, dst, zrows):
    nparts = len(en_parts)
    pe = en_parts[0].shape[0]
    rows_per_w = pe // _NW
    np_ = _NP
    zn = np_ // _NSUB

    @functools.partial(
        pl.kernel,
        out_type=jax.ShapeDtypeStruct((_NCORES, np_, H), jnp.float32),
        mesh=_sc_mesh(),
        scratch_types=[
            pltpu.VMEM((_CHUNK,), jnp.int32),
            pltpu.VMEM((_CHUNK, H), jnp.float32),
            pltpu.VMEM((_CHUNK,), jnp.int32),
            pltpu.VMEM((_CHUNK, H), jnp.float32),
            pltpu.VMEM_SHARED((np_, H), jnp.float32),
            pltpu.SemaphoreType.DMA,
            pltpu.SemaphoreType.DMA,
        ],
    )
    def k(*refs):
        en_hbms = refs[:nparts]
        dst_hbm, z_hbm, out_hbm, i0, v0, i1, v1, acc_sh, l0, l1 = refs[nparts:]
        cid = lax.axis_index("c")
        sid = lax.axis_index("s")
        wid = sid * _NCORES + cid
        pltpu.sync_copy(z_hbm, acc_sh.at[pl.ds(sid * zn, zn)])
        plsc.subcore_barrier()

        sets = ((i0, v0, l0), (i1, v1, l1))
        nch = rows_per_w // _CHUNK
        base = wid * rows_per_w

        def fire_load(p, off, s):
            idx_v, buf_v, lsem = sets[s]
            pltpu.async_copy(dst_hbm.at[pl.ds(p * pe + off, _CHUNK)], idx_v, lsem)
            pltpu.async_copy(en_hbms[p].at[pl.ds(off, _CHUNK)], buf_v, lsem)

        def finish(p, s):
            idx_v, buf_v, lsem = sets[s]
            pltpu.make_async_copy(dst_hbm.at[pl.ds(base, _CHUNK)], idx_v,
                                  lsem).wait()
            pltpu.make_async_copy(en_hbms[p].at[pl.ds(base, _CHUNK)], buf_v,
                                  lsem).wait()
            pltpu.sync_copy(buf_v, acc_sh.at[idx_v], add=True)

        for p in range(nparts):
            fire_load(p, base, 0)

            @pl.loop(0, (nch - 1) // 2)
            def _(jj):
                off = base + jj * (2 * _CHUNK)
                fire_load(p, off + _CHUNK, 1)
                finish(p, 0)
                fire_load(p, off + 2 * _CHUNK, 0)
                finish(p, 1)

            finish(p, 0)

        plsc.subcore_barrier()
        pltpu.sync_copy(acc_sh.at[pl.ds(sid * zn, zn)],
                        out_hbm.at[cid].at[pl.ds(sid * zn, zn)])

    return k(*en_parts, dst, zrows)


def kernel(x, edge_attr, params, edge_index):
    src = edge_index[0]
    dst = edge_index[1]

    h = _mlp_ln(x, params["ne"], BN)
    e_parts = [_mlp_ln(edge_attr, params["ee"], BE, row0=p * _PE, rows=_PE)
               for p in range(_NPARTS)]

    zrows = jnp.zeros((_NP // _NSUB, H), jnp.float32)
    for bi, blk in enumerate(params["mp"]):
        w1 = blk["edge"]["W1"]
        ps, pd = _proj(h, w1[:H], w1[H:2 * H])
        en_parts = []
        for p in range(_NPARTS):
            gs, gd = _sc_gather_pair(ps, pd, src, dst, p * _PE, _PE)
            if bi < 2:
                en_p, e_parts[p] = _edge_update(gs, gd, e_parts[p], w1[2 * H:],
                                                blk["edge"], residual=True)
            else:
                (en_p,) = _edge_update(gs, gd, e_parts[p], w1[2 * H:],
                                       blk["edge"], residual=False)
            en_parts.append(en_p)
        parts = _sc_segsum(en_parts, dst, zrows)
        h = _node_update(h, parts[0, :N], parts[1, :N], blk["node"])

    mask = (jnp.arange(_DOUT) % (1 + 2 * Z) >= 1 + Z).astype(jnp.float32).reshape(1, _DOUT)
    raw = _pool_head(h, params["gate_W"].reshape(1, H), params["head"], mask)
    raw = raw.reshape(1, K, 1 + 2 * Z)
    return raw[:, :, 0], raw[:, :, 1:1 + Z], raw[:, :, 1 + Z:]
